# baseline (speedup 1.0000x reference)
import jax
import jax.numpy as jnp
from jax import lax
from jax.experimental import pallas as pl
from jax.experimental.pallas import tpu as pltpu
from jax.experimental.pallas import tpu_sc as plsc

_B = 16384
_D = 64
_NC = 2
_NS = 16
_NW = _NC * _NS
_BPW = _B // _NW
_CHUNK = 128
_NCHUNK = _BPW // _CHUNK
_G = 16
_NG = _BPW // _G


def _body(featT_hbm, lab_hbm, cent2_hbm, out_hbm, idx_v, row_v, feat_v,
          gath_v, part_v, fsem, gsem):
    wid = lax.axis_index("s") * _NC + lax.axis_index("c")
    base = wid * _BPW

    pltpu.sync_copy(lab_hbm.at[pl.ds(base, _BPW)], idx_v)
    fcopy = pltpu.async_copy(featT_hbm.at[:, pl.ds(base, _BPW)], feat_v, fsem)

    def mk_rows(i, _):
        for j in range(_NCHUNK):
            row_v[j, pl.ds(i * 16, 16)] = (
                idx_v[pl.ds(j * _CHUNK + i * 16, 16)] >> 1)
        return 0

    lax.fori_loop(0, _CHUNK // 16, mk_rows, 0)

    for j in range(_NCHUNK):
        pltpu.async_copy(
            cent2_hbm.at[row_v.at[j]],
            gath_v.at[pl.ds(j * _CHUNK, _CHUNK)], gsem)
    for j in range(_NCHUNK):
        pltpu.make_async_copy(
            cent2_hbm.at[pl.ds(0, _CHUNK)],
            gath_v.at[pl.ds(0, _CHUNK)], gsem).wait()
    fcopy.wait()

    iota = lax.iota(jnp.int32, 16)

    def group_body(g, accs):
        chunk = idx_v[pl.ds(g * _G, _G)]
        offv = (chunk & 1) << 6
        a0, a1, a2, a3 = accs
        for b in range(_G):
            k = g * _G + b
            off = offv[b]
            crow = jnp.full((16,), k, jnp.int32)
            cbase = iota + off
            fcol = jnp.full((16,), k, jnp.int32)
            cv0 = plsc.load_gather(gath_v, [crow, cbase])
            fv0 = plsc.load_gather(feat_v, [iota, fcol])
            cv1 = plsc.load_gather(gath_v, [crow, cbase + 16])
            fv1 = plsc.load_gather(feat_v, [iota + 16, fcol])
            cv2 = plsc.load_gather(gath_v, [crow, cbase + 32])
            fv2 = plsc.load_gather(feat_v, [iota + 32, fcol])
            cv3 = plsc.load_gather(gath_v, [crow, cbase + 48])
            fv3 = plsc.load_gather(feat_v, [iota + 48, fcol])
            d0 = fv0 - cv0
            d1 = fv1 - cv1
            d2 = fv2 - cv2
            d3 = fv3 - cv3
            a0 = a0 + d0 * d0
            a1 = a1 + d1 * d1
            a2 = a2 + d2 * d2
            a3 = a3 + d3 * d3
        return (a0, a1, a2, a3)

    zero = jnp.zeros((16,), jnp.float32)
    accs = lax.fori_loop(0, _NG, group_body, (zero, zero, zero, zero))

    part_v[...] = (accs[0] + accs[1]) + (accs[2] + accs[3])
    pltpu.sync_copy(part_v, out_hbm.at[wid])


@jax.jit
def kernel(features, labels, centers):
    featT = features.T
    cent2 = centers.reshape(500000, 128)
    labels = labels.astype(jnp.int32)
    mesh = plsc.VectorSubcoreMesh(core_axis_name="c", subcore_axis_name="s")
    partials = pl.kernel(
        _body,
        out_type=jax.ShapeDtypeStruct((_NW, 16), jnp.float32),
        mesh=mesh,
        scratch_types=[
            pltpu.VMEM((_BPW,), jnp.int32),
            pltpu.VMEM((_NCHUNK, _CHUNK), jnp.int32),
            pltpu.VMEM((_D, _BPW), jnp.float32),
            pltpu.VMEM((_BPW, _CHUNK), jnp.float32),
            pltpu.VMEM((16,), jnp.float32),
            pltpu.SemaphoreType.DMA,
            pltpu.SemaphoreType.DMA,
        ],
        compiler_params=pltpu.CompilerParams(
            use_tc_tiling_on_sc=True, needs_layout_passes=False),
    )(featT, labels, cent2)
    return jnp.sum(partials) * (1.0 / (_B * _D))


# speedup vs baseline: 2.4602x; 2.4602x over previous
import jax
import jax.numpy as jnp
from jax import lax
from jax.experimental import pallas as pl
from jax.experimental.pallas import tpu as pltpu
from jax.experimental.pallas import tpu_sc as plsc

_B = 16384
_D = 64
_V = 1000000
_NC = 2
_NS = 16
_NW = _NC * _NS
_BPW = _B // _NW
_NT = (_V + 127) // 128
_SHARD = 245
_LAST_T = _NT - 1
_LAST_OFF = _LAST_T * 128


def _extract_body(lab_hbm, centT_hbm, ext_hbm, lab_all, listb, sortb, hist,
                  rptr, wptr, slab, slab_last, stage, bidx, lsem, ssem, wsem):
    wid = lax.axis_index("s") * _NC + lax.axis_index("c")
    lo_t = wid * _SHARD
    hi_t = lo_t + _SHARD
    reg_cnt = jnp.clip(_LAST_T - lo_t, 0, _SHARD)

    pltpu.sync_copy(lab_hbm, lab_all.at[pl.ds(0, _B)])

    iota = lax.iota(jnp.int32, 16)
    lane0 = iota == 0
    ones = jnp.full((16,), 1, jnp.int32)

    def scan_body(i, ptr):
        ch = lab_all[pl.ds(i * 16, 16)]
        tc = ch >> 7
        m = (tc >= lo_t) & (tc < hi_t)
        plsc.store_compressed(listb.at[pl.ds(ptr, 16)], iota + i * 16, mask=m)
        return ptr + plsc.all_reduce_population_count(m)[0]

    ent = lax.fori_loop(0, _B // 16, scan_body, 0)

    def zero_body(j, _):
        wptr[pl.ds(j * 16, 16)] = jnp.zeros((16,), jnp.int32)
        hist[pl.ds(j * 16, 16)] = jnp.zeros((16,), jnp.int32)
        return 0

    lax.fori_loop(0, 16, zero_body, 0)

    def count_body(e, _):
        b = listb[pl.ds(e, 16)][0]
        c = lab_all[pl.ds(b, 16)][0]
        tcl = (c >> 7) - lo_t
        plsc.addupdate_scatter(hist, [jnp.full((16,), tcl, jnp.int32)], ones,
                               mask=lane0)
        return 0

    lax.fori_loop(0, ent, count_body, 0)

    def prefix_body(j, s):
        ch = hist[pl.ds(j * 16, 16)]
        cs = plsc.cumsum(ch)
        ex = (cs - ch) + s
        rptr[pl.ds(j * 16, 16)] = ex
        wptr[pl.ds(j * 16, 16)] = ex
        return s + cs[15]

    lax.fori_loop(0, 16, prefix_body, 0)

    def place_body(e, _):
        b = listb[pl.ds(e, 16)][0]
        c = lab_all[pl.ds(b, 16)][0]
        tcl = (c >> 7) - lo_t
        tsp = jnp.full((16,), tcl, jnp.int32)
        slot = plsc.load_gather(wptr, [tsp])[0]
        plsc.store_scatter(sortb, [jnp.full((16,), slot, jnp.int32)],
                           jnp.full((16,), b, jnp.int32), mask=lane0)
        plsc.addupdate_scatter(wptr, [tsp], ones, mask=lane0)
        return 0

    lax.fori_loop(0, ent, place_body, 0)

    def issue(t, bank):
        off = pl.multiple_of((lo_t + t) * 128, 128)
        pltpu.async_copy(centT_hbm.at[:, pl.ds(off, 128)], slab.at[bank],
                         ssem.at[bank])

    def wait_slab(bank):
        pltpu.make_async_copy(centT_hbm.at[:, pl.ds(0, 128)], slab.at[bank],
                              ssem.at[bank]).wait()

    def wait_ring(slot):
        pltpu.make_async_copy(stage.at[0], ext_hbm.at[pl.ds(0, 1)],
                              wsem.at[slot]).wait()

    def extract_one(e, q, src_ref, width):
        slot = q & 3

        @pl.when(q >= 4)
        def _():
            wait_ring(slot)

        b = sortb[pl.ds(e, 16)][0]
        c = lab_all[pl.ds(b, 16)][0]
        l = c & 127
        lsp = jnp.full((16,), l, jnp.int32)
        for g in range(4):
            cv = plsc.load_gather(src_ref, [iota + 16 * g, lsp])
            stage[slot, 0, pl.ds(16 * g, 16)] = cv
        plsc.store_scatter(bidx, [jnp.full((16,), slot, jnp.int32),
                                  jnp.zeros((16,), jnp.int32)],
                           jnp.full((16,), b, jnp.int32), mask=lane0)
        pltpu.async_copy(stage.at[slot], ext_hbm.at[bidx.at[slot]],
                         wsem.at[slot])
        return q + 1

    issue(0, 0)

    def tile_body(t, q):
        bank = t & 1

        @pl.when(t + 1 < reg_cnt)
        def _():
            issue(t + 1, 1 - bank)

        wait_slab(bank)
        e0 = plsc.load_gather(rptr, [jnp.full((16,), t, jnp.int32)])[0]
        n = plsc.load_gather(hist, [jnp.full((16,), t, jnp.int32)])[0]

        def inner(i, qq):
            return extract_one(e0 + i, qq, slab.at[bank], 128)

        return lax.fori_loop(0, n, inner, q)

    q = lax.fori_loop(0, reg_cnt, tile_body, 0)

    trel_last = _LAST_T - 31 * _SHARD

    def last_pass(q):
        pltpu.sync_copy(centT_hbm.at[:, pl.ds(_LAST_OFF, 64)], slab_last)
        e0 = plsc.load_gather(rptr, [jnp.full((16,), trel_last, jnp.int32)])[0]
        n = plsc.load_gather(hist, [jnp.full((16,), trel_last, jnp.int32)])[0]

        def inner(i, qq):
            return extract_one(e0 + i, qq, slab_last, 64)

        return lax.fori_loop(0, n, inner, q)

    q = lax.cond(wid == 31, last_pass, lambda x: x, q)

    def drain(i, _):
        wait_ring(i & 3)
        return 0

    lax.fori_loop(0, jnp.minimum(q, 4), drain, 0)


def _mse_body(featT_hbm, ext_hbm, out_hbm, feat_v, ext_v, part_v, fsem, esem):
    wid = lax.axis_index("s") * _NC + lax.axis_index("c")
    base = wid * _BPW
    fcopy = pltpu.async_copy(featT_hbm.at[:, pl.ds(base, _BPW)], feat_v, fsem)
    pltpu.async_copy(ext_hbm.at[pl.ds(base, _BPW)], ext_v, esem).wait()
    fcopy.wait()

    iota = lax.iota(jnp.int32, 16)

    def group_body(g, accs):
        a0, a1, a2, a3 = accs
        for b in range(16):
            k = g * 16 + b
            fcol = jnp.full((16,), k, jnp.int32)
            cv0 = ext_v[k, pl.ds(0, 16)]
            fv0 = plsc.load_gather(feat_v, [iota, fcol])
            cv1 = ext_v[k, pl.ds(16, 16)]
            fv1 = plsc.load_gather(feat_v, [iota + 16, fcol])
            cv2 = ext_v[k, pl.ds(32, 16)]
            fv2 = plsc.load_gather(feat_v, [iota + 32, fcol])
            cv3 = ext_v[k, pl.ds(48, 16)]
            fv3 = plsc.load_gather(feat_v, [iota + 48, fcol])
            d0 = fv0 - cv0
            d1 = fv1 - cv1
            d2 = fv2 - cv2
            d3 = fv3 - cv3
            a0 = a0 + d0 * d0
            a1 = a1 + d1 * d1
            a2 = a2 + d2 * d2
            a3 = a3 + d3 * d3
        return (a0, a1, a2, a3)

    zero = jnp.zeros((16,), jnp.float32)
    accs = lax.fori_loop(0, _BPW // 16, group_body, (zero, zero, zero, zero))
    part_v[...] = (accs[0] + accs[1]) + (accs[2] + accs[3])
    pltpu.sync_copy(part_v, out_hbm.at[wid])


@jax.jit
def kernel(features, labels, centers):
    featT = features.T
    centT = centers.T
    labels = labels.astype(jnp.int32)
    mesh = plsc.VectorSubcoreMesh(core_axis_name="c", subcore_axis_name="s")
    params = pltpu.CompilerParams(use_tc_tiling_on_sc=True,
                                  needs_layout_passes=False)
    ext = pl.kernel(
        _extract_body,
        out_type=jax.ShapeDtypeStruct((_B, 128), jnp.float32),
        mesh=mesh,
        scratch_types=[
            pltpu.VMEM((_B + 16, ), jnp.int32),
            pltpu.VMEM((_B + 16, ), jnp.int32),
            pltpu.VMEM((_B + 16, ), jnp.int32),
            pltpu.VMEM((272, ), jnp.int32),
            pltpu.VMEM((272, ), jnp.int32),
            pltpu.VMEM((272, ), jnp.int32),
            pltpu.VMEM((2, _D, 128), jnp.float32),
            pltpu.VMEM((_D, 64), jnp.float32),
            pltpu.VMEM((4, 1, 128), jnp.float32),
            pltpu.VMEM((4, 1), jnp.int32),
            pltpu.SemaphoreType.DMA,
            pltpu.SemaphoreType.DMA((2, )),
            pltpu.SemaphoreType.DMA((4, )),
        ],
        compiler_params=params,
    )(labels, centT)
    partials = pl.kernel(
        _mse_body,
        out_type=jax.ShapeDtypeStruct((_NW, 16), jnp.float32),
        mesh=mesh,
        scratch_types=[
            pltpu.VMEM((_D, _BPW), jnp.float32),
            pltpu.VMEM((_BPW, 128), jnp.float32),
            pltpu.VMEM((16, ), jnp.float32),
            pltpu.SemaphoreType.DMA,
            pltpu.SemaphoreType.DMA,
        ],
        compiler_params=params,
    )(featT, ext)
    return jnp.sum(partials) * (1.0 / (_B * _D))


# speedup vs baseline: 3.1807x; 1.2928x over previous
import jax
import jax.numpy as jnp
from jax import lax
from jax.experimental import pallas as pl
from jax.experimental.pallas import tpu as pltpu
from jax.experimental.pallas import tpu_sc as plsc

_B = 16384
_D = 64
_V = 1000000
_NC = 2
_NS = 16
_NW = _NC * _NS
_BPW = _B // _NW
_NT = (_V + 127) // 128
_SHARD = 245
_LAST_T = _NT - 1
_LAST_OFF = _LAST_T * 128


def _extract_body(lab_hbm, centT_hbm, ext_hbm, lab_all, listb, sortb, hist,
                  rptr, wptr, slab, slab_last, stage, bidx, lsem, ssem, wsem):
    wid = lax.axis_index("s") * _NC + lax.axis_index("c")
    lo_t = wid * _SHARD
    hi_t = lo_t + _SHARD
    reg_cnt = jnp.clip(_LAST_T - lo_t, 0, _SHARD)

    def issue(t, bank):
        off = pl.multiple_of((lo_t + t) * 128, 128)
        pltpu.async_copy(centT_hbm.at[:, pl.ds(off, 128)], slab.at[bank],
                         ssem.at[bank])

    for r in range(3):
        @pl.when(r < reg_cnt)
        def _():
            issue(r, r)

    pltpu.sync_copy(lab_hbm, lab_all.at[pl.ds(0, _B)])

    iota = lax.iota(jnp.int32, 16)
    lane0 = iota == 0
    ones = jnp.full((16,), 1, jnp.int32)

    def scan_body(i, ptr):
        ch = lab_all[pl.ds(i * 16, 16)]
        tc = ch >> 7
        m = (tc >= lo_t) & (tc < hi_t)
        plsc.store_compressed(listb.at[pl.ds(ptr, 16)], iota + i * 16, mask=m)
        return ptr + plsc.all_reduce_population_count(m)[0]

    ent = lax.fori_loop(0, _B // 16, scan_body, 0)

    def zero_body(j, _):
        wptr[pl.ds(j * 16, 16)] = jnp.zeros((16,), jnp.int32)
        hist[pl.ds(j * 16, 16)] = jnp.zeros((16,), jnp.int32)
        return 0

    lax.fori_loop(0, 16, zero_body, 0)

    def count_body(e, _):
        b = listb[pl.ds(e, 16)][0]
        c = lab_all[pl.ds(b, 16)][0]
        tcl = (c >> 7) - lo_t
        plsc.addupdate_scatter(hist, [jnp.full((16,), tcl, jnp.int32)], ones,
                               mask=lane0)
        return 0

    lax.fori_loop(0, ent, count_body, 0)

    def prefix_body(j, s):
        ch = hist[pl.ds(j * 16, 16)]
        cs = plsc.cumsum(ch)
        ex = (cs - ch) + s
        rptr[pl.ds(j * 16, 16)] = ex
        wptr[pl.ds(j * 16, 16)] = ex
        return s + cs[15]

    lax.fori_loop(0, 16, prefix_body, 0)

    def place_body(e, _):
        b = listb[pl.ds(e, 16)][0]
        c = lab_all[pl.ds(b, 16)][0]
        tcl = (c >> 7) - lo_t
        tsp = jnp.full((16,), tcl, jnp.int32)
        slot = plsc.load_gather(wptr, [tsp])[0]
        plsc.store_scatter(sortb, [jnp.full((16,), slot, jnp.int32)],
                           jnp.full((16,), b, jnp.int32), mask=lane0)
        plsc.addupdate_scatter(wptr, [tsp], ones, mask=lane0)
        return 0

    lax.fori_loop(0, ent, place_body, 0)

    def wait_slab(bank):
        pltpu.make_async_copy(centT_hbm.at[:, pl.ds(0, 128)], slab.at[bank],
                              ssem.at[bank]).wait()

    def wait_ring(slot):
        pltpu.make_async_copy(stage.at[0], ext_hbm.at[pl.ds(0, 1)],
                              wsem.at[slot]).wait()

    def extract_one(e, q, src_ref, width):
        slot = q & 3

        @pl.when(q >= 4)
        def _():
            wait_ring(slot)

        b = sortb[pl.ds(e, 16)][0]
        c = lab_all[pl.ds(b, 16)][0]
        l = c & 127
        lsp = jnp.full((16,), l, jnp.int32)
        for g in range(4):
            cv = plsc.load_gather(src_ref, [iota + 16 * g, lsp])
            stage[slot, 0, pl.ds(16 * g, 16)] = cv
        plsc.store_scatter(bidx, [jnp.full((16,), slot, jnp.int32),
                                  jnp.zeros((16,), jnp.int32)],
                           jnp.full((16,), b, jnp.int32), mask=lane0)
        pltpu.async_copy(stage.at[slot], ext_hbm.at[bidx.at[slot]],
                         wsem.at[slot])
        return q + 1

    def tile_body(t, q):
        bank = t & 3

        @pl.when(t + 3 < reg_cnt)
        def _():
            issue(t + 3, (t + 3) & 3)

        wait_slab(bank)
        e0 = plsc.load_gather(rptr, [jnp.full((16,), t, jnp.int32)])[0]
        n = plsc.load_gather(hist, [jnp.full((16,), t, jnp.int32)])[0]

        def inner(i, qq):
            return extract_one(e0 + i, qq, slab.at[bank], 128)

        return lax.fori_loop(0, n, inner, q)

    q = lax.fori_loop(0, reg_cnt, tile_body, 0)

    trel_last = _LAST_T - 31 * _SHARD

    def last_pass(q):
        pltpu.sync_copy(centT_hbm.at[:, pl.ds(_LAST_OFF, 64)], slab_last)
        e0 = plsc.load_gather(rptr, [jnp.full((16,), trel_last, jnp.int32)])[0]
        n = plsc.load_gather(hist, [jnp.full((16,), trel_last, jnp.int32)])[0]

        def inner(i, qq):
            return extract_one(e0 + i, qq, slab_last, 64)

        return lax.fori_loop(0, n, inner, q)

    q = lax.cond(wid == 31, last_pass, lambda x: x, q)

    def drain(i, _):
        wait_ring(i & 3)
        return 0

    lax.fori_loop(0, jnp.minimum(q, 4), drain, 0)


def _mse_body(featT_hbm, ext_hbm, out_hbm, feat_v, ext_v, part_v, fsem, esem):
    wid = lax.axis_index("s") * _NC + lax.axis_index("c")
    base = wid * _BPW
    fcopy = pltpu.async_copy(featT_hbm.at[:, pl.ds(base, _BPW)], feat_v, fsem)
    pltpu.async_copy(ext_hbm.at[pl.ds(base, _BPW)], ext_v, esem).wait()
    fcopy.wait()

    iota = lax.iota(jnp.int32, 16)

    def group_body(g, accs):
        a0, a1, a2, a3 = accs
        for b in range(16):
            k = g * 16 + b
            fcol = jnp.full((16,), k, jnp.int32)
            cv0 = ext_v[k, pl.ds(0, 16)]
            fv0 = plsc.load_gather(feat_v, [iota, fcol])
            cv1 = ext_v[k, pl.ds(16, 16)]
            fv1 = plsc.load_gather(feat_v, [iota + 16, fcol])
            cv2 = ext_v[k, pl.ds(32, 16)]
            fv2 = plsc.load_gather(feat_v, [iota + 32, fcol])
            cv3 = ext_v[k, pl.ds(48, 16)]
            fv3 = plsc.load_gather(feat_v, [iota + 48, fcol])
            d0 = fv0 - cv0
            d1 = fv1 - cv1
            d2 = fv2 - cv2
            d3 = fv3 - cv3
            a0 = a0 + d0 * d0
            a1 = a1 + d1 * d1
            a2 = a2 + d2 * d2
            a3 = a3 + d3 * d3
        return (a0, a1, a2, a3)

    zero = jnp.zeros((16,), jnp.float32)
    accs = lax.fori_loop(0, _BPW // 16, group_body, (zero, zero, zero, zero))
    part_v[...] = (accs[0] + accs[1]) + (accs[2] + accs[3])
    pltpu.sync_copy(part_v, out_hbm.at[wid])


@jax.jit
def kernel(features, labels, centers):
    featT = features.T
    centT = centers.T
    labels = labels.astype(jnp.int32)
    mesh = plsc.VectorSubcoreMesh(core_axis_name="c", subcore_axis_name="s")
    params = pltpu.CompilerParams(use_tc_tiling_on_sc=True,
                                  needs_layout_passes=False)
    ext = pl.kernel(
        _extract_body,
        out_type=jax.ShapeDtypeStruct((_B, 128), jnp.float32),
        mesh=mesh,
        scratch_types=[
            pltpu.VMEM((_B + 16, ), jnp.int32),
            pltpu.VMEM((_B + 16, ), jnp.int32),
            pltpu.VMEM((_B + 16, ), jnp.int32),
            pltpu.VMEM((272, ), jnp.int32),
            pltpu.VMEM((272, ), jnp.int32),
            pltpu.VMEM((272, ), jnp.int32),
            pltpu.VMEM((4, _D, 128), jnp.float32),
            pltpu.VMEM((_D, 64), jnp.float32),
            pltpu.VMEM((4, 1, 128), jnp.float32),
            pltpu.VMEM((4, 1), jnp.int32),
            pltpu.SemaphoreType.DMA,
            pltpu.SemaphoreType.DMA((4, )),
            pltpu.SemaphoreType.DMA((4, )),
        ],
        compiler_params=params,
    )(labels, centT)
    partials = pl.kernel(
        _mse_body,
        out_type=jax.ShapeDtypeStruct((_NW, 16), jnp.float32),
        mesh=mesh,
        scratch_types=[
            pltpu.VMEM((_D, _BPW), jnp.float32),
            pltpu.VMEM((_BPW, 128), jnp.float32),
            pltpu.VMEM((16, ), jnp.float32),
            pltpu.SemaphoreType.DMA,
            pltpu.SemaphoreType.DMA,
        ],
        compiler_params=params,
    )(featT, ext)
    return jnp.sum(partials) * (1.0 / (_B * _D))


# speedup vs baseline: 3.4358x; 1.0802x over previous
import jax
import jax.numpy as jnp
from jax import lax
from jax.experimental import pallas as pl
from jax.experimental.pallas import tpu as pltpu
from jax.experimental.pallas import tpu_sc as plsc

_B = 16384
_D = 64
_V = 1000000
_NC = 2
_NS = 16
_NW = _NC * _NS
_BPW = _B // _NW
_NT = (_V + 127) // 128
_SHARD = 245
_LAST_T = _NT - 1
_LAST_OFF = _LAST_T * 128


def _extract_body(lab_hbm, centT_hbm, ext_hbm, lab_all, listb, sortb, hist,
                  rptr, wptr, slab, slab_last, stage, bidx, lsem, ssem, wsem):
    wid = lax.axis_index("s") * _NC + lax.axis_index("c")
    lo_t = wid * _SHARD
    hi_t = lo_t + _SHARD
    reg_cnt = jnp.clip(_LAST_T - lo_t, 0, _SHARD)

    def issue(t, bank):
        off = pl.multiple_of((lo_t + t) * 128, 128)
        pltpu.async_copy(centT_hbm.at[:, pl.ds(off, 128)], slab.at[bank],
                         ssem.at[bank])

    for r in range(7):
        @pl.when(r < reg_cnt)
        def _():
            issue(r, r)

    pltpu.sync_copy(lab_hbm, lab_all.at[pl.ds(0, _B)])

    iota = lax.iota(jnp.int32, 16)
    lane0 = iota == 0
    ones = jnp.full((16,), 1, jnp.int32)

    def scan_body(i, ptr):
        ch = lab_all[pl.ds(i * 16, 16)]
        tc = ch >> 7
        m = (tc >= lo_t) & (tc < hi_t)
        plsc.store_compressed(listb.at[pl.ds(ptr, 16)], iota + i * 16, mask=m)
        return ptr + plsc.all_reduce_population_count(m)[0]

    ent = lax.fori_loop(0, _B // 16, scan_body, 0)

    def zero_body(j, _):
        wptr[pl.ds(j * 16, 16)] = jnp.zeros((16,), jnp.int32)
        hist[pl.ds(j * 16, 16)] = jnp.zeros((16,), jnp.int32)
        return 0

    lax.fori_loop(0, 16, zero_body, 0)

    def count_body(e, _):
        b = listb[pl.ds(e, 16)][0]
        c = lab_all[pl.ds(b, 16)][0]
        tcl = (c >> 7) - lo_t
        plsc.addupdate_scatter(hist, [jnp.full((16,), tcl, jnp.int32)], ones,
                               mask=lane0)
        return 0

    lax.fori_loop(0, ent, count_body, 0)

    def prefix_body(j, s):
        ch = hist[pl.ds(j * 16, 16)]
        cs = plsc.cumsum(ch)
        ex = (cs - ch) + s
        rptr[pl.ds(j * 16, 16)] = ex
        wptr[pl.ds(j * 16, 16)] = ex
        return s + cs[15]

    lax.fori_loop(0, 16, prefix_body, 0)

    def place_body(e, _):
        b = listb[pl.ds(e, 16)][0]
        c = lab_all[pl.ds(b, 16)][0]
        tcl = (c >> 7) - lo_t
        tsp = jnp.full((16,), tcl, jnp.int32)
        slot = plsc.load_gather(wptr, [tsp])[0]
        plsc.store_scatter(sortb, [jnp.full((16,), slot, jnp.int32)],
                           jnp.full((16,), b, jnp.int32), mask=lane0)
        plsc.addupdate_scatter(wptr, [tsp], ones, mask=lane0)
        return 0

    lax.fori_loop(0, ent, place_body, 0)

    def wait_slab(bank):
        pltpu.make_async_copy(centT_hbm.at[:, pl.ds(0, 128)], slab.at[bank],
                              ssem.at[bank]).wait()

    def wait_ring(slot):
        pltpu.make_async_copy(stage.at[0], ext_hbm.at[pl.ds(0, 1)],
                              wsem.at[slot]).wait()

    def extract_one(e, q, src_ref, width):
        slot = q & 3

        @pl.when(q >= 4)
        def _():
            wait_ring(slot)

        b = sortb[pl.ds(e, 16)][0]
        c = lab_all[pl.ds(b, 16)][0]
        l = c & 127
        lsp = jnp.full((16,), l, jnp.int32)
        for g in range(4):
            cv = plsc.load_gather(src_ref, [iota + 16 * g, lsp])
            stage[slot, 0, pl.ds(16 * g, 16)] = cv
        plsc.store_scatter(bidx, [jnp.full((16,), slot, jnp.int32),
                                  jnp.zeros((16,), jnp.int32)],
                           jnp.full((16,), b, jnp.int32), mask=lane0)
        pltpu.async_copy(stage.at[slot], ext_hbm.at[bidx.at[slot]],
                         wsem.at[slot])
        return q + 1

    def tile_body(t, q):
        bank = t & 7

        @pl.when(t + 7 < reg_cnt)
        def _():
            issue(t + 7, (t + 7) & 7)

        wait_slab(bank)
        e0 = plsc.load_gather(rptr, [jnp.full((16,), t, jnp.int32)])[0]
        n = plsc.load_gather(hist, [jnp.full((16,), t, jnp.int32)])[0]

        def inner(i, qq):
            return extract_one(e0 + i, qq, slab.at[bank], 128)

        return lax.fori_loop(0, n, inner, q)

    q = lax.fori_loop(0, reg_cnt, tile_body, 0)

    trel_last = _LAST_T - 31 * _SHARD

    def last_pass(q):
        pltpu.sync_copy(centT_hbm.at[:, pl.ds(_LAST_OFF, 64)], slab_last)
        e0 = plsc.load_gather(rptr, [jnp.full((16,), trel_last, jnp.int32)])[0]
        n = plsc.load_gather(hist, [jnp.full((16,), trel_last, jnp.int32)])[0]

        def inner(i, qq):
            return extract_one(e0 + i, qq, slab_last, 64)

        return lax.fori_loop(0, n, inner, q)

    q = lax.cond(wid == 31, last_pass, lambda x: x, q)

    def drain(i, _):
        wait_ring(i & 3)
        return 0

    lax.fori_loop(0, jnp.minimum(q, 4), drain, 0)


def _mse_body(featT_hbm, ext_hbm, out_hbm, feat_v, ext_v, part_v, fsem, esem):
    wid = lax.axis_index("s") * _NC + lax.axis_index("c")
    base = wid * _BPW
    fcopy = pltpu.async_copy(featT_hbm.at[:, pl.ds(base, _BPW)], feat_v, fsem)
    pltpu.async_copy(ext_hbm.at[pl.ds(base, _BPW)], ext_v, esem).wait()
    fcopy.wait()

    iota = lax.iota(jnp.int32, 16)

    def group_body(g, accs):
        a0, a1, a2, a3 = accs
        for b in range(16):
            k = g * 16 + b
            fcol = jnp.full((16,), k, jnp.int32)
            cv0 = ext_v[k, pl.ds(0, 16)]
            fv0 = plsc.load_gather(feat_v, [iota, fcol])
            cv1 = ext_v[k, pl.ds(16, 16)]
            fv1 = plsc.load_gather(feat_v, [iota + 16, fcol])
            cv2 = ext_v[k, pl.ds(32, 16)]
            fv2 = plsc.load_gather(feat_v, [iota + 32, fcol])
            cv3 = ext_v[k, pl.ds(48, 16)]
            fv3 = plsc.load_gather(feat_v, [iota + 48, fcol])
            d0 = fv0 - cv0
            d1 = fv1 - cv1
            d2 = fv2 - cv2
            d3 = fv3 - cv3
            a0 = a0 + d0 * d0
            a1 = a1 + d1 * d1
            a2 = a2 + d2 * d2
            a3 = a3 + d3 * d3
        return (a0, a1, a2, a3)

    zero = jnp.zeros((16,), jnp.float32)
    accs = lax.fori_loop(0, _BPW // 16, group_body, (zero, zero, zero, zero))
    part_v[...] = (accs[0] + accs[1]) + (accs[2] + accs[3])
    pltpu.sync_copy(part_v, out_hbm.at[wid])


@jax.jit
def kernel(features, labels, centers):
    featT = features.T
    centT = centers.T
    labels = labels.astype(jnp.int32)
    mesh = plsc.VectorSubcoreMesh(core_axis_name="c", subcore_axis_name="s")
    params = pltpu.CompilerParams(use_tc_tiling_on_sc=True,
                                  needs_layout_passes=False)
    ext = pl.kernel(
        _extract_body,
        out_type=jax.ShapeDtypeStruct((_B, 128), jnp.float32),
        mesh=mesh,
        scratch_types=[
            pltpu.VMEM((_B + 16, ), jnp.int32),
            pltpu.VMEM((_B + 16, ), jnp.int32),
            pltpu.VMEM((_B + 16, ), jnp.int32),
            pltpu.VMEM((272, ), jnp.int32),
            pltpu.VMEM((272, ), jnp.int32),
            pltpu.VMEM((272, ), jnp.int32),
            pltpu.VMEM((8, _D, 128), jnp.float32),
            pltpu.VMEM((_D, 64), jnp.float32),
            pltpu.VMEM((4, 1, 128), jnp.float32),
            pltpu.VMEM((4, 1), jnp.int32),
            pltpu.SemaphoreType.DMA,
            pltpu.SemaphoreType.DMA((8, )),
            pltpu.SemaphoreType.DMA((4, )),
        ],
        compiler_params=params,
    )(labels, centT)
    partials = pl.kernel(
        _mse_body,
        out_type=jax.ShapeDtypeStruct((_NW, 16), jnp.float32),
        mesh=mesh,
        scratch_types=[
            pltpu.VMEM((_D, _BPW), jnp.float32),
            pltpu.VMEM((_BPW, 128), jnp.float32),
            pltpu.VMEM((16, ), jnp.float32),
            pltpu.SemaphoreType.DMA,
            pltpu.SemaphoreType.DMA,
        ],
        compiler_params=params,
    )(featT, ext)
    return jnp.sum(partials) * (1.0 / (_B * _D))
